# baseline (speedup 1.0000x reference)
import jax
import jax.numpy as jnp
from jax import lax
from jax.experimental import pallas as pl
from jax.experimental.pallas import tpu as pltpu
from jax.experimental.pallas import tpu_sc as plsc

NC = 2
NS = 16
NW = NC * NS
L = 16
BATCH = 16384
DIM = 16
BPW = BATCH // NW
NCHUNK = 4
CHUNK = BPW // NCHUNK


def _body(u_hbm, i_hbm, ut_hbm, it_hbm, out_hbm,
          idx_u, idx_i, ue, ie, out_v, *sems):
    wid = lax.axis_index("s") * NC + lax.axis_index("c")
    base = wid * BPW

    pltpu.sync_copy(u_hbm.at[wid], idx_u)
    pltpu.sync_copy(i_hbm.at[wid], idx_i)

    def fire_chunk(j, buf):
        def enq(g, carry):
            r0 = g * L
            iu_vec = idx_u[pl.ds(j * CHUNK + r0, L)]
            ii_vec = idx_i[pl.ds(j * CHUNK + r0, L)]
            for l in range(L):
                pltpu.async_copy(ut_hbm.at[iu_vec[l]],
                                 ue.at[buf, r0 + l], sems[l])
                pltpu.async_copy(it_hbm.at[ii_vec[l]],
                                 ie.at[buf, r0 + l], sems[l])
            return carry

        lax.fori_loop(0, CHUNK // L, enq, 0)

    def drain_chunk(buf):
        per = CHUNK // L
        for l in range(L):
            pltpu.make_async_copy(ut_hbm.at[pl.ds(0, per)],
                                  ue.at[buf, pl.ds(0, per)], sems[l]).wait()
            pltpu.make_async_copy(it_hbm.at[pl.ds(0, per)],
                                  ie.at[buf, pl.ds(0, per)], sems[l]).wait()

    def compute_chunk(j, buf):
        def group(g, carry):
            r0 = g * L
            rows = lax.iota(jnp.int32, L) + r0
            acc = jnp.zeros((L,), jnp.float32)
            for d in range(DIM):
                col = jnp.full((L,), d, jnp.int32)
                uc = plsc.load_gather(ue.at[buf], [rows, col])
                ic = plsc.load_gather(ie.at[buf], [rows, col])
                acc = acc + uc * ic
            out_v[pl.ds(j * CHUNK + r0, L)] = acc
            return carry

        lax.fori_loop(0, CHUNK // L, group, 0)

    fire_chunk(0, 0)
    for j in range(NCHUNK):
        drain_chunk(j % 2)
        if j + 1 < NCHUNK:
            fire_chunk(j + 1, (j + 1) % 2)
        compute_chunk(j, j % 2)

    pltpu.sync_copy(out_v, out_hbm.at[pl.ds(base, BPW)])


@jax.jit
def kernel(u, i, user_table, item_table):
    u2 = u.astype(jnp.int32).reshape(NW, BPW)
    i2 = i.astype(jnp.int32).reshape(NW, BPW)
    mesh = plsc.VectorSubcoreMesh(core_axis_name="c", subcore_axis_name="s")
    f = pl.kernel(
        _body,
        out_type=jax.ShapeDtypeStruct((BATCH,), jnp.float32),
        mesh=mesh,
        compiler_params=pltpu.CompilerParams(needs_layout_passes=False),
        scratch_types=[
            pltpu.VMEM((BPW,), jnp.int32),
            pltpu.VMEM((BPW,), jnp.int32),
            pltpu.VMEM((2, CHUNK, DIM), jnp.float32),
            pltpu.VMEM((2, CHUNK, DIM), jnp.float32),
            pltpu.VMEM((BPW,), jnp.float32),
        ] + [pltpu.SemaphoreType.DMA] * L,
    )
    return f(u2, i2, user_table, item_table)


# speedup vs baseline: 1.0616x; 1.0616x over previous
import jax
import jax.numpy as jnp
from jax import lax
from jax.experimental import pallas as pl
from jax.experimental.pallas import tpu as pltpu
from jax.experimental.pallas import tpu_sc as plsc

NC = 2
NS = 16
NW = NC * NS
L = 16
BATCH = 16384
DIM = 16
BPW = BATCH // NW
NCHUNK = 4
CHUNK = BPW // NCHUNK


def _body(u_hbm, i_hbm, ut_hbm, it_hbm, out_hbm,
          idx_u, idx_i, ue, ie, out_v, *sems):
    wid = lax.axis_index("s") * NC + lax.axis_index("c")
    base = wid * BPW

    pltpu.sync_copy(u_hbm.at[wid], idx_u)
    pltpu.sync_copy(i_hbm.at[wid], idx_i)

    def fire_chunk(j, buf):
        def enq(g, carry):
            r0 = g * L
            iu_vec = idx_u[pl.ds(j * CHUNK + r0, L)]
            ii_vec = idx_i[pl.ds(j * CHUNK + r0, L)]
            for l in range(L):
                pltpu.async_copy(ut_hbm.at[iu_vec[l]],
                                 ue.at[buf, r0 + l], sems[0])
                pltpu.async_copy(it_hbm.at[ii_vec[l]],
                                 ie.at[buf, r0 + l], sems[1])
            return carry

        lax.fori_loop(0, CHUNK // L, enq, 0)

    def drain_chunk(buf):
        pltpu.make_async_copy(ut_hbm.at[pl.ds(0, CHUNK)],
                              ue.at[buf], sems[0]).wait()
        pltpu.make_async_copy(it_hbm.at[pl.ds(0, CHUNK)],
                              ie.at[buf], sems[1]).wait()

    def compute_chunk(j, buf):
        def group(g, carry):
            r0 = g * L
            rows = lax.iota(jnp.int32, L) + r0
            acc = jnp.zeros((L,), jnp.float32)
            for d in range(DIM):
                col = jnp.full((L,), d, jnp.int32)
                uc = plsc.load_gather(ue.at[buf], [rows, col])
                ic = plsc.load_gather(ie.at[buf], [rows, col])
                acc = acc + uc * ic
            out_v[pl.ds(j * CHUNK + r0, L)] = acc
            return carry

        lax.fori_loop(0, CHUNK // L, group, 0)

    fire_chunk(0, 0)
    for j in range(NCHUNK):
        drain_chunk(j % 2)
        if j + 1 < NCHUNK:
            fire_chunk(j + 1, (j + 1) % 2)
        compute_chunk(j, j % 2)

    pltpu.sync_copy(out_v, out_hbm.at[pl.ds(base, BPW)])


@jax.jit
def kernel(u, i, user_table, item_table):
    u2 = u.astype(jnp.int32).reshape(NW, BPW)
    i2 = i.astype(jnp.int32).reshape(NW, BPW)
    mesh = plsc.VectorSubcoreMesh(core_axis_name="c", subcore_axis_name="s")
    f = pl.kernel(
        _body,
        out_type=jax.ShapeDtypeStruct((BATCH,), jnp.float32),
        mesh=mesh,
        compiler_params=pltpu.CompilerParams(needs_layout_passes=False),
        scratch_types=[
            pltpu.VMEM((BPW,), jnp.int32),
            pltpu.VMEM((BPW,), jnp.int32),
            pltpu.VMEM((2, CHUNK, DIM), jnp.float32),
            pltpu.VMEM((2, CHUNK, DIM), jnp.float32),
            pltpu.VMEM((BPW,), jnp.float32),
        ] + [pltpu.SemaphoreType.DMA] * 2,
    )
    return f(u2, i2, user_table, item_table)
